# baseline (speedup 1.0000x reference)
import jax
import jax.numpy as jnp
from jax import lax
from jax.experimental import pallas as pl
from jax.experimental.pallas import tpu as pltpu
from jax.experimental.pallas import tpu_sc as plsc

N, E, D, G = 10000, 320000, 128, 64
NC, NS = 2, 16
NW = NC * NS
CH = 128
NCH = 80
EPW = NCH * CH
EPAD = NW * EPW
EPW_REAL = E // NW
NACC = N + CH
ZR = 624
ZTAIL = N - NS * ZR
BR = 1000

_mesh = plsc.VectorSubcoreMesh(
    core_axis_name="c", subcore_axis_name="s", num_cores=NC, num_subcores=NS)


def _deg_body(dst_hbm, out_hbm, idx_v, deg_v):
    wid = lax.axis_index("c") * NS + lax.axis_index("s")
    pltpu.sync_copy(dst_hbm.at[pl.ds(wid * EPW_REAL, EPW_REAL)], idx_v)

    def zero(i, _):
        deg_v[pl.ds(i * 16, 16)] = jnp.zeros((16,), jnp.float32)
        return 0
    lax.fori_loop(0, N // 16, zero, 0)

    ones = jnp.ones((16,), jnp.float32)

    def add(i, _):
        plsc.addupdate_scatter(deg_v, [idx_v[pl.ds(i * 16, 16)]], ones)
        return 0
    lax.fori_loop(0, EPW_REAL // 16, add, 0)
    pltpu.sync_copy(deg_v, out_hbm.at[wid])


def _deg_call(dst):
    k = pl.kernel(
        _deg_body,
        out_type=jax.ShapeDtypeStruct((NW, N), jnp.float32),
        mesh=_mesh,
        compiler_params=pltpu.CompilerParams(needs_layout_passes=False),
        scratch_types=[
            pltpu.VMEM((EPW_REAL,), jnp.int32),
            pltpu.VMEM((N,), jnp.float32),
        ],
    )
    return k(dst)


def _msg_body(ts_hbm, ec_hbm, out0, out1,
              eb0, eb1, rows0, rows1, shared, sg0, sg1, se0, se1):
    core = lax.axis_index("c")
    sub = lax.axis_index("s")
    wid = core * NS + sub

    def zrow(i, _):
        for j in range(8):
            rows0[i, pl.ds(j * 16, 16)] = jnp.zeros((16,), jnp.float32)
        return 0
    lax.fori_loop(0, CH, zrow, 0)
    zbase = sub * ZR
    for k in range(4):
        pltpu.sync_copy(rows0, shared.at[pl.ds(zbase + k * CH, CH)])
    pltpu.sync_copy(rows0.at[pl.ds(0, ZR - 4 * CH)],
                    shared.at[pl.ds(zbase + 4 * CH, ZR - 4 * CH)])

    @pl.when(sub == NS - 1)
    def _():
        pltpu.sync_copy(rows0.at[pl.ds(0, ZTAIL)],
                        shared.at[pl.ds(NS * ZR, ZTAIL)])
    plsc.subcore_barrier()

    cid = wid * NCH

    pltpu.sync_copy(ec_hbm.at[cid], eb0)
    pltpu.async_copy(ts_hbm.at[eb0.at[0]], rows0, sg0)
    pltpu.async_copy(ec_hbm.at[cid + 1], eb1, se1)

    def step(i, _):
        a0 = cid + 2 * i
        pltpu.make_async_copy(ec_hbm.at[a0 + 1], eb1, se1).wait()
        pltpu.async_copy(ts_hbm.at[eb1.at[0]], rows1, sg1)
        pltpu.make_async_copy(ts_hbm.at[eb0.at[0]], rows0, sg0).wait()
        pltpu.sync_copy(rows0, shared.at[eb0.at[1]], add=True)
        pltpu.async_copy(ec_hbm.at[a0 + 2], eb0, se0)
        pltpu.make_async_copy(ec_hbm.at[a0 + 2], eb0, se0).wait()
        pltpu.async_copy(ts_hbm.at[eb0.at[0]], rows0, sg0)
        pltpu.make_async_copy(ts_hbm.at[eb1.at[0]], rows1, sg1).wait()
        pltpu.sync_copy(rows1, shared.at[eb1.at[1]], add=True)
        pltpu.async_copy(ec_hbm.at[a0 + 3], eb1, se1)
        return 0
    lax.fori_loop(0, NCH // 2 - 1, step, 0)

    pltpu.make_async_copy(ec_hbm.at[cid + NCH - 1], eb1, se1).wait()
    pltpu.async_copy(ts_hbm.at[eb1.at[0]], rows1, sg1)
    pltpu.make_async_copy(ts_hbm.at[eb0.at[0]], rows0, sg0).wait()
    pltpu.sync_copy(rows0, shared.at[eb0.at[1]], add=True)
    pltpu.make_async_copy(ts_hbm.at[eb1.at[0]], rows1, sg1).wait()
    pltpu.sync_copy(rows1, shared.at[eb1.at[1]], add=True)

    plsc.subcore_barrier()
    rs = sub * ZR

    @pl.when(core == 0)
    def _():
        pltpu.sync_copy(shared.at[pl.ds(rs, ZR)], out0.at[pl.ds(rs, ZR)])

        @pl.when(sub == NS - 1)
        def _():
            pltpu.sync_copy(shared.at[pl.ds(NS * ZR, ZTAIL)],
                            out0.at[pl.ds(NS * ZR, ZTAIL)])

    @pl.when(core == 1)
    def _():
        pltpu.sync_copy(shared.at[pl.ds(rs, ZR)], out1.at[pl.ds(rs, ZR)])

        @pl.when(sub == NS - 1)
        def _():
            pltpu.sync_copy(shared.at[pl.ds(NS * ZR, ZTAIL)],
                            out1.at[pl.ds(NS * ZR, ZTAIL)])


def _msg_call(ts, echunks):
    k = pl.kernel(
        _msg_body,
        out_type=(jax.ShapeDtypeStruct((N, D), jnp.float32),
                  jax.ShapeDtypeStruct((N, D), jnp.float32)),
        mesh=_mesh,
        scratch_types=[
            pltpu.VMEM((2, CH), jnp.int32),
            pltpu.VMEM((2, CH), jnp.int32),
            pltpu.VMEM((CH, D), jnp.float32),
            pltpu.VMEM((CH, D), jnp.float32),
            pltpu.VMEM_SHARED((NACC, D), jnp.float32),
            pltpu.SemaphoreType.DMA,
            pltpu.SemaphoreType.DMA,
            pltpu.SemaphoreType.DMA,
            pltpu.SemaphoreType.DMA,
        ],
    )
    return k(ts, echunks)


def _dinvpre_call(degpT, x, W1):
    def body(d_ref, x_ref, w_ref, dv_ref, ts_ref):
        deg = jnp.sum(d_ref[...], axis=1, keepdims=True) + 1.0
        dv = jnp.broadcast_to(lax.rsqrt(deg), (BR, D))
        dv_ref[...] = dv
        t = jnp.dot(x_ref[...], w_ref[...], preferred_element_type=jnp.float32)
        ts_ref[...] = t * dv
    return pl.pallas_call(
        body,
        out_shape=(jax.ShapeDtypeStruct((N, D), jnp.float32),
                   jax.ShapeDtypeStruct((N, D), jnp.float32)),
        grid=(N // BR,),
        in_specs=[pl.BlockSpec((BR, NW), lambda i: (i, 0)),
                  pl.BlockSpec((BR, D), lambda i: (i, 0)),
                  pl.BlockSpec((D, D), lambda i: (0, 0))],
        out_specs=(pl.BlockSpec((BR, D), lambda i: (i, 0)),
                   pl.BlockSpec((BR, D), lambda i: (i, 0))),
    )(degpT, x, W1)


def _postpre_call(p0, p1, ts, dinvb, b2d, res, Wn):
    spec = pl.BlockSpec((BR, D), lambda i: (i, 0))
    if res is None:
        def body(p0r, p1r, tsr, dvr, br, wr, hr, tr):
            dv = dvr[...]
            h = jnp.maximum((p0r[...] + p1r[...] + tsr[...]) * dv + br[...],
                            0.0)
            hr[...] = h
            tr[...] = jnp.dot(h, wr[...],
                              preferred_element_type=jnp.float32) * dv
        args = (p0, p1, ts, dinvb, b2d, Wn)
        extra = []
    else:
        def body(p0r, p1r, tsr, dvr, br, rr, wr, hr, tr):
            dv = dvr[...]
            h = jnp.maximum((p0r[...] + p1r[...] + tsr[...]) * dv + br[...],
                            0.0) + rr[...]
            hr[...] = h
            tr[...] = jnp.dot(h, wr[...],
                              preferred_element_type=jnp.float32) * dv
        args = (p0, p1, ts, dinvb, b2d, res, Wn)
        extra = [spec]
    return pl.pallas_call(
        body,
        out_shape=(jax.ShapeDtypeStruct((N, D), jnp.float32),
                   jax.ShapeDtypeStruct((N, D), jnp.float32)),
        grid=(N // BR,),
        in_specs=[spec, spec, spec, spec,
                  pl.BlockSpec((1, D), lambda i: (0, 0))] + extra
                 + [pl.BlockSpec((D, D), lambda i: (0, 0))],
        out_specs=(spec, spec),
    )(*args)


def _post_call(p0, p1, ts, dinvb, b2d, res):
    def body(p0r, p1r, tsr, dvr, br, rr, o_ref):
        s = (p0r[...] + p1r[...] + tsr[...]) * dvr[...] + br[...]
        o_ref[...] = jnp.maximum(s, 0.0) + rr[...]
    spec = pl.BlockSpec((BR, D), lambda i: (i, 0))
    return pl.pallas_call(
        body,
        out_shape=jax.ShapeDtypeStruct((N, D), jnp.float32),
        grid=(N // BR,),
        in_specs=[spec, spec, spec, spec,
                  pl.BlockSpec((1, D), lambda i: (0, 0)), spec],
        out_specs=spec,
    )(p0, p1, ts, dinvb, b2d, res)


def _pool_mlp_call(h, batchb, batch_row, fW1, fb1, fW2, fb2, fW3, fb3):
    def body(h_ref, bb_ref, br_ref, w1_ref, c1_ref, w2_ref, c2_ref,
             w3_ref, c3_ref, o_ref, mxs_ref):
        hv = h_ref[...]
        bb = bb_ref[...]
        seg = lax.broadcasted_iota(jnp.int32, (G, N), 0)
        bcast = jnp.broadcast_to(br_ref[...], (G, N))
        Mt = (seg == bcast).astype(jnp.float32)
        sums = jnp.dot(Mt, hv, preferred_element_type=jnp.float32)
        cnts = jnp.dot(Mt, jnp.ones((N, 1), jnp.float32),
                       preferred_element_type=jnp.float32)
        mean = sums / jnp.maximum(cnts, 1.0)

        def mx(g, _):
            m = bb == g
            mxrow = jnp.max(jnp.where(m, hv, -jnp.inf), axis=0, keepdims=True)
            mxs_ref[pl.ds(g, 1), :] = mxrow
            return 0
        lax.fori_loop(0, G, mx, 0)
        xmax = mxs_ref[...]

        w1 = w1_ref[...]
        a = jnp.dot(mean, w1[:D, :], preferred_element_type=jnp.float32)
        a = a + jnp.dot(xmax, w1[D:, :], preferred_element_type=jnp.float32)
        a = jnp.maximum(a + c1_ref[...], 0.0)
        a = jnp.maximum(
            jnp.dot(a, w2_ref[...], preferred_element_type=jnp.float32)
            + c2_ref[...], 0.0)
        o_ref[...] = (jnp.dot(a, w3_ref[...], preferred_element_type=jnp.float32)
                      + c3_ref[...])

    return pl.pallas_call(
        body,
        out_shape=jax.ShapeDtypeStruct((G, 1), jnp.float32),
        scratch_shapes=[pltpu.VMEM((G, D), jnp.float32)],
    )(h, batchb, batch_row, fW1, fb1, fW2, fb2, fW3, fb3)


def kernel(x, edge_index, batch, W1, b1, W2, b2, W3, b3,
           fW1, fb1, fW2, fb2, fW3, fb3):
    dst = edge_index[1]

    pad = jnp.stack([jnp.zeros((EPAD - E,), jnp.int32),
                     N + (jnp.arange(EPAD - E, dtype=jnp.int32) % CH)])
    echunks = (jnp.concatenate([edge_index, pad], axis=1)
               .reshape(2, NW, NCH, CH).transpose(1, 2, 0, 3)
               .reshape(NW * NCH, 2, CH))

    degp = _deg_call(dst)
    batchb = jnp.broadcast_to(batch[:, None], (N, D))
    batch_row = batch[None, :]

    dinvb, ts1 = _dinvpre_call(degp.T, x, W1)
    p0, p1 = _msg_call(ts1, echunks)
    h1, ts2 = _postpre_call(p0, p1, ts1, dinvb, b1.reshape(1, D), None, W2)
    p0, p1 = _msg_call(ts2, echunks)
    h2, ts3 = _postpre_call(p0, p1, ts2, dinvb, b2.reshape(1, D), h1, W3)
    p0, p1 = _msg_call(ts3, echunks)
    h3 = _post_call(p0, p1, ts3, dinvb, b3.reshape(1, D), h2)

    return _pool_mlp_call(h3, batchb, batch_row, fW1, fb1.reshape(1, D),
                          fW2, fb2.reshape(1, D // 2), fW3, fb3.reshape(1, 1))


# speedup vs baseline: 2.8284x; 2.8284x over previous
import jax
import jax.numpy as jnp
from jax import lax
from jax.experimental import pallas as pl
from jax.experimental.pallas import tpu as pltpu
from jax.experimental.pallas import tpu_sc as plsc

N, E, D, G = 10000, 320000, 128, 64
NC, NS = 2, 16
NW = NC * NS
CH = 128
NCH = 80
EPW = NCH * CH
EPAD = NW * EPW
EPW_REAL = E // NW
NACC = N + CH
ZR = 624
ZTAIL = N - NS * ZR
BR = 1000

_mesh = plsc.VectorSubcoreMesh(
    core_axis_name="c", subcore_axis_name="s", num_cores=NC, num_subcores=NS)


def _deg_body(dst_hbm, out_hbm, idx_v, deg_v):
    wid = lax.axis_index("c") * NS + lax.axis_index("s")
    pltpu.sync_copy(dst_hbm.at[pl.ds(wid * EPW_REAL, EPW_REAL)], idx_v)

    def zero(i, _):
        deg_v[pl.ds(i * 16, 16)] = jnp.zeros((16,), jnp.float32)
        return 0
    lax.fori_loop(0, N // 16, zero, 0)

    ones = jnp.ones((16,), jnp.float32)

    def add(i, _):
        plsc.addupdate_scatter(deg_v, [idx_v[pl.ds(i * 16, 16)]], ones)
        return 0
    lax.fori_loop(0, EPW_REAL // 16, add, 0)
    pltpu.sync_copy(deg_v, out_hbm.at[wid])


def _deg_call(dst):
    k = pl.kernel(
        _deg_body,
        out_type=jax.ShapeDtypeStruct((NW, N), jnp.float32),
        mesh=_mesh,
        compiler_params=pltpu.CompilerParams(needs_layout_passes=False),
        scratch_types=[
            pltpu.VMEM((EPW_REAL,), jnp.int32),
            pltpu.VMEM((N,), jnp.float32),
        ],
    )
    return k(dst)


def _msg_body(ts_hbm, ec_hbm, out0, out1,
              eb0, eb1, rows0, rows1, shared, sg0, sg1, se0, se1):
    core = lax.axis_index("c")
    sub = lax.axis_index("s")
    wid = core * NS + sub

    def zrow(i, _):
        for j in range(8):
            rows0[i, pl.ds(j * 16, 16)] = jnp.zeros((16,), jnp.float32)
        return 0
    lax.fori_loop(0, CH, zrow, 0)
    zbase = sub * ZR
    for k in range(4):
        pltpu.sync_copy(rows0, shared.at[pl.ds(zbase + k * CH, CH)])
    pltpu.sync_copy(rows0.at[pl.ds(0, ZR - 4 * CH)],
                    shared.at[pl.ds(zbase + 4 * CH, ZR - 4 * CH)])

    @pl.when(sub == NS - 1)
    def _():
        pltpu.sync_copy(rows0.at[pl.ds(0, ZTAIL)],
                        shared.at[pl.ds(NS * ZR, ZTAIL)])
        pltpu.sync_copy(rows0, shared.at[pl.ds(N, CH)])
    plsc.subcore_barrier()

    cid = wid * NCH

    pltpu.sync_copy(ec_hbm.at[cid], eb0)
    pltpu.async_copy(ts_hbm.at[eb0.at[0]], rows0, sg0)
    pltpu.async_copy(ec_hbm.at[cid + 1], eb1, se1)

    def step(i, _):
        a0 = cid + 2 * i
        pltpu.make_async_copy(ec_hbm.at[a0 + 1], eb1, se1).wait()
        pltpu.async_copy(ts_hbm.at[eb1.at[0]], rows1, sg1)
        pltpu.make_async_copy(ts_hbm.at[eb0.at[0]], rows0, sg0).wait()
        pltpu.sync_copy(rows0, shared.at[eb0.at[1]], add=True)
        pltpu.async_copy(ec_hbm.at[a0 + 2], eb0, se0)
        pltpu.make_async_copy(ec_hbm.at[a0 + 2], eb0, se0).wait()
        pltpu.async_copy(ts_hbm.at[eb0.at[0]], rows0, sg0)
        pltpu.make_async_copy(ts_hbm.at[eb1.at[0]], rows1, sg1).wait()
        pltpu.sync_copy(rows1, shared.at[eb1.at[1]], add=True)
        pltpu.async_copy(ec_hbm.at[a0 + 3], eb1, se1)
        return 0
    lax.fori_loop(0, NCH // 2 - 1, step, 0)

    pltpu.make_async_copy(ec_hbm.at[cid + NCH - 1], eb1, se1).wait()
    pltpu.async_copy(ts_hbm.at[eb1.at[0]], rows1, sg1)
    pltpu.make_async_copy(ts_hbm.at[eb0.at[0]], rows0, sg0).wait()
    pltpu.sync_copy(rows0, shared.at[eb0.at[1]], add=True)
    pltpu.make_async_copy(ts_hbm.at[eb1.at[0]], rows1, sg1).wait()
    pltpu.sync_copy(rows1, shared.at[eb1.at[1]], add=True)

    plsc.subcore_barrier()
    rs = sub * ZR

    @pl.when(core == 0)
    def _():
        pltpu.sync_copy(shared.at[pl.ds(rs, ZR)], out0.at[pl.ds(rs, ZR)])

        @pl.when(sub == NS - 1)
        def _():
            pltpu.sync_copy(shared.at[pl.ds(NS * ZR, ZTAIL)],
                            out0.at[pl.ds(NS * ZR, ZTAIL)])

    @pl.when(core == 1)
    def _():
        pltpu.sync_copy(shared.at[pl.ds(rs, ZR)], out1.at[pl.ds(rs, ZR)])

        @pl.when(sub == NS - 1)
        def _():
            pltpu.sync_copy(shared.at[pl.ds(NS * ZR, ZTAIL)],
                            out1.at[pl.ds(NS * ZR, ZTAIL)])


def _msg_call(ts, echunks):
    k = pl.kernel(
        _msg_body,
        out_type=(jax.ShapeDtypeStruct((N, D), jnp.float32),
                  jax.ShapeDtypeStruct((N, D), jnp.float32)),
        mesh=_mesh,
        scratch_types=[
            pltpu.VMEM((2, CH), jnp.int32),
            pltpu.VMEM((2, CH), jnp.int32),
            pltpu.VMEM((CH, D), jnp.float32),
            pltpu.VMEM((CH, D), jnp.float32),
            pltpu.VMEM_SHARED((NACC, D), jnp.float32),
            pltpu.SemaphoreType.DMA,
            pltpu.SemaphoreType.DMA,
            pltpu.SemaphoreType.DMA,
            pltpu.SemaphoreType.DMA,
        ],
    )
    return k(ts, echunks)


def _dinvpre_call(degpT, x, W1):
    def body(d_ref, x_ref, w_ref, dv_ref, ts_ref):
        deg = jnp.sum(d_ref[...], axis=1, keepdims=True) + 1.0
        dv = jnp.broadcast_to(lax.rsqrt(deg), (BR, D))
        dv_ref[...] = dv
        t = jnp.dot(x_ref[...], w_ref[...], preferred_element_type=jnp.float32)
        ts_ref[...] = t * dv
    return pl.pallas_call(
        body,
        out_shape=(jax.ShapeDtypeStruct((N, D), jnp.float32),
                   jax.ShapeDtypeStruct((N, D), jnp.float32)),
        grid=(N // BR,),
        in_specs=[pl.BlockSpec((BR, NW), lambda i: (i, 0)),
                  pl.BlockSpec((BR, D), lambda i: (i, 0)),
                  pl.BlockSpec((D, D), lambda i: (0, 0))],
        out_specs=(pl.BlockSpec((BR, D), lambda i: (i, 0)),
                   pl.BlockSpec((BR, D), lambda i: (i, 0))),
    )(degpT, x, W1)


def _postpre_call(p0, p1, ts, dinvb, b2d, res, Wn):
    spec = pl.BlockSpec((BR, D), lambda i: (i, 0))
    if res is None:
        def body(p0r, p1r, tsr, dvr, br, wr, hr, tr):
            dv = dvr[...]
            h = jnp.maximum((p0r[...] + p1r[...] + tsr[...]) * dv + br[...],
                            0.0)
            hr[...] = h
            tr[...] = jnp.dot(h, wr[...],
                              preferred_element_type=jnp.float32) * dv
        args = (p0, p1, ts, dinvb, b2d, Wn)
        extra = []
    else:
        def body(p0r, p1r, tsr, dvr, br, rr, wr, hr, tr):
            dv = dvr[...]
            h = jnp.maximum((p0r[...] + p1r[...] + tsr[...]) * dv + br[...],
                            0.0) + rr[...]
            hr[...] = h
            tr[...] = jnp.dot(h, wr[...],
                              preferred_element_type=jnp.float32) * dv
        args = (p0, p1, ts, dinvb, b2d, res, Wn)
        extra = [spec]
    return pl.pallas_call(
        body,
        out_shape=(jax.ShapeDtypeStruct((N, D), jnp.float32),
                   jax.ShapeDtypeStruct((N, D), jnp.float32)),
        grid=(N // BR,),
        in_specs=[spec, spec, spec, spec,
                  pl.BlockSpec((1, D), lambda i: (0, 0))] + extra
                 + [pl.BlockSpec((D, D), lambda i: (0, 0))],
        out_specs=(spec, spec),
    )(*args)


def _post_call(p0, p1, ts, dinvb, b2d, res):
    def body(p0r, p1r, tsr, dvr, br, rr, o_ref):
        s = (p0r[...] + p1r[...] + tsr[...]) * dvr[...] + br[...]
        o_ref[...] = jnp.maximum(s, 0.0) + rr[...]
    spec = pl.BlockSpec((BR, D), lambda i: (i, 0))
    return pl.pallas_call(
        body,
        out_shape=jax.ShapeDtypeStruct((N, D), jnp.float32),
        grid=(N // BR,),
        in_specs=[spec, spec, spec, spec,
                  pl.BlockSpec((1, D), lambda i: (0, 0)), spec],
        out_specs=spec,
    )(p0, p1, ts, dinvb, b2d, res)


def _pool_mlp_call(h, batchb, batch_row, fW1, fb1, fW2, fb2, fW3, fb3):
    def body(h_ref, bb_ref, br_ref, w1_ref, c1_ref, w2_ref, c2_ref,
             w3_ref, c3_ref, o_ref, mxs_ref):
        hv = h_ref[...]
        bb = bb_ref[...]
        seg = lax.broadcasted_iota(jnp.int32, (G, N), 0)
        bcast = jnp.broadcast_to(br_ref[...], (G, N))
        Mt = (seg == bcast).astype(jnp.float32)
        sums = jnp.dot(Mt, hv, preferred_element_type=jnp.float32)
        cnts = jnp.dot(Mt, jnp.ones((N, 1), jnp.float32),
                       preferred_element_type=jnp.float32)
        mean = sums / jnp.maximum(cnts, 1.0)

        def mx(g, _):
            m = bb == g
            mxrow = jnp.max(jnp.where(m, hv, -jnp.inf), axis=0, keepdims=True)
            mxs_ref[pl.ds(g, 1), :] = mxrow
            return 0
        lax.fori_loop(0, G, mx, 0)
        xmax = mxs_ref[...]

        w1 = w1_ref[...]
        a = jnp.dot(mean, w1[:D, :], preferred_element_type=jnp.float32)
        a = a + jnp.dot(xmax, w1[D:, :], preferred_element_type=jnp.float32)
        a = jnp.maximum(a + c1_ref[...], 0.0)
        a = jnp.maximum(
            jnp.dot(a, w2_ref[...], preferred_element_type=jnp.float32)
            + c2_ref[...], 0.0)
        o_ref[...] = (jnp.dot(a, w3_ref[...], preferred_element_type=jnp.float32)
                      + c3_ref[...])

    return pl.pallas_call(
        body,
        out_shape=jax.ShapeDtypeStruct((G, 1), jnp.float32),
        scratch_shapes=[pltpu.VMEM((G, D), jnp.float32)],
    )(h, batchb, batch_row, fW1, fb1, fW2, fb2, fW3, fb3)


def kernel(x, edge_index, batch, W1, b1, W2, b2, W3, b3,
           fW1, fb1, fW2, fb2, fW3, fb3):
    dst = edge_index[1]

    ppw = EPW - EPW_REAL
    pad_src = jnp.broadcast_to(jnp.arange(ppw, dtype=jnp.int32), (NW, ppw))
    pad_dst = jnp.broadcast_to(N + (jnp.arange(ppw, dtype=jnp.int32) % CH),
                               (NW, ppw))
    pad = jnp.stack([pad_src, pad_dst])
    echunks = (jnp.concatenate([edge_index.reshape(2, NW, EPW_REAL), pad],
                               axis=2)
               .reshape(2, NW, NCH, CH).transpose(1, 2, 0, 3)
               .reshape(NW * NCH, 2, CH))

    degp = _deg_call(dst)
    batchb = jnp.broadcast_to(batch[:, None], (N, D))
    batch_row = batch[None, :]

    dinvb, ts1 = _dinvpre_call(degp.T, x, W1)
    p0, p1 = _msg_call(ts1, echunks)
    h1, ts2 = _postpre_call(p0, p1, ts1, dinvb, b1.reshape(1, D), None, W2)
    p0, p1 = _msg_call(ts2, echunks)
    h2, ts3 = _postpre_call(p0, p1, ts2, dinvb, b2.reshape(1, D), h1, W3)
    p0, p1 = _msg_call(ts3, echunks)
    h3 = _post_call(p0, p1, ts3, dinvb, b3.reshape(1, D), h2)

    return _pool_mlp_call(h3, batchb, batch_row, fW1, fb1.reshape(1, D),
                          fW2, fb2.reshape(1, D // 2), fW3, fb3.reshape(1, 1))


# speedup vs baseline: 3.2195x; 1.1383x over previous
import jax
import jax.numpy as jnp
from jax import lax
from jax.experimental import pallas as pl
from jax.experimental.pallas import tpu as pltpu
from jax.experimental.pallas import tpu_sc as plsc

N, E, D, G = 10000, 320000, 128, 64
NC, NS = 2, 16
NW = NC * NS
CH = 128
NCH = 80
EPW = NCH * CH
EPAD = NW * EPW
EPW_REAL = E // NW
NACC = N + CH
ZR = 624
ZTAIL = N - NS * ZR
BR = 1000

_mesh = plsc.VectorSubcoreMesh(
    core_axis_name="c", subcore_axis_name="s", num_cores=NC, num_subcores=NS)


def _deg_body(dst_hbm, out_hbm, idx_v, deg_v):
    wid = lax.axis_index("c") * NS + lax.axis_index("s")
    pltpu.sync_copy(dst_hbm.at[pl.ds(wid * EPW_REAL, EPW_REAL)], idx_v)

    def zero(i, _):
        deg_v[pl.ds(i * 16, 16)] = jnp.zeros((16,), jnp.float32)
        return 0
    lax.fori_loop(0, N // 16, zero, 0)

    ones = jnp.ones((16,), jnp.float32)

    def add(i, _):
        plsc.addupdate_scatter(deg_v, [idx_v[pl.ds(i * 16, 16)]], ones)
        return 0
    lax.fori_loop(0, EPW_REAL // 16, add, 0)
    pltpu.sync_copy(deg_v, out_hbm.at[wid])


def _deg_call(dst):
    k = pl.kernel(
        _deg_body,
        out_type=jax.ShapeDtypeStruct((NW, N), jnp.float32),
        mesh=_mesh,
        compiler_params=pltpu.CompilerParams(needs_layout_passes=False),
        scratch_types=[
            pltpu.VMEM((EPW_REAL,), jnp.int32),
            pltpu.VMEM((N,), jnp.float32),
        ],
    )
    return k(dst)


def _msg_body(ts_hbm, ec_hbm, out0, out1,
              eb0, eb1, rows0, rows1, shared, sg0, sg1, se0, se1):
    core = lax.axis_index("c")
    sub = lax.axis_index("s")
    wid = core * NS + sub

    def zrow(i, _):
        for j in range(8):
            rows0[i, pl.ds(j * 16, 16)] = jnp.zeros((16,), jnp.float32)
        return 0
    lax.fori_loop(0, CH, zrow, 0)
    zbase = sub * ZR
    for k in range(4):
        pltpu.sync_copy(rows0, shared.at[pl.ds(zbase + k * CH, CH)])
    pltpu.sync_copy(rows0.at[pl.ds(0, ZR - 4 * CH)],
                    shared.at[pl.ds(zbase + 4 * CH, ZR - 4 * CH)])

    @pl.when(sub == NS - 1)
    def _():
        pltpu.sync_copy(rows0.at[pl.ds(0, ZTAIL)],
                        shared.at[pl.ds(NS * ZR, ZTAIL)])
        pltpu.sync_copy(rows0, shared.at[pl.ds(N, CH)])
    plsc.subcore_barrier()

    cid = wid * NCH

    pltpu.sync_copy(ec_hbm.at[cid], eb0)
    pltpu.async_copy(ts_hbm.at[eb0.at[0]], rows0, sg0)
    pltpu.async_copy(ec_hbm.at[cid + 1], eb1, se1)

    def step(i, _):
        a0 = cid + 2 * i
        pltpu.make_async_copy(ec_hbm.at[a0 + 1], eb1, se1).wait()
        pltpu.async_copy(ts_hbm.at[eb1.at[0]], rows1, sg1)
        pltpu.make_async_copy(ts_hbm.at[eb0.at[0]], rows0, sg0).wait()
        pltpu.sync_copy(rows0, shared.at[eb0.at[1]], add=True)
        pltpu.async_copy(ec_hbm.at[a0 + 2], eb0, se0)
        pltpu.make_async_copy(ec_hbm.at[a0 + 2], eb0, se0).wait()
        pltpu.async_copy(ts_hbm.at[eb0.at[0]], rows0, sg0)
        pltpu.make_async_copy(ts_hbm.at[eb1.at[0]], rows1, sg1).wait()
        pltpu.sync_copy(rows1, shared.at[eb1.at[1]], add=True)
        pltpu.async_copy(ec_hbm.at[a0 + 3], eb1, se1)
        return 0
    lax.fori_loop(0, NCH // 2 - 1, step, 0)

    pltpu.make_async_copy(ec_hbm.at[cid + NCH - 1], eb1, se1).wait()
    pltpu.async_copy(ts_hbm.at[eb1.at[0]], rows1, sg1)
    pltpu.make_async_copy(ts_hbm.at[eb0.at[0]], rows0, sg0).wait()
    pltpu.sync_copy(rows0, shared.at[eb0.at[1]], add=True)
    pltpu.make_async_copy(ts_hbm.at[eb1.at[0]], rows1, sg1).wait()
    pltpu.sync_copy(rows1, shared.at[eb1.at[1]], add=True)

    plsc.subcore_barrier()
    rs = sub * ZR

    @pl.when(core == 0)
    def _():
        pltpu.sync_copy(shared.at[pl.ds(rs, ZR)], out0.at[pl.ds(rs, ZR)])

        @pl.when(sub == NS - 1)
        def _():
            pltpu.sync_copy(shared.at[pl.ds(NS * ZR, ZTAIL)],
                            out0.at[pl.ds(NS * ZR, ZTAIL)])

    @pl.when(core == 1)
    def _():
        pltpu.sync_copy(shared.at[pl.ds(rs, ZR)], out1.at[pl.ds(rs, ZR)])

        @pl.when(sub == NS - 1)
        def _():
            pltpu.sync_copy(shared.at[pl.ds(NS * ZR, ZTAIL)],
                            out1.at[pl.ds(NS * ZR, ZTAIL)])


def _msg_call(ts, echunks):
    k = pl.kernel(
        _msg_body,
        out_type=(jax.ShapeDtypeStruct((N, D), jnp.float32),
                  jax.ShapeDtypeStruct((N, D), jnp.float32)),
        mesh=_mesh,
        scratch_types=[
            pltpu.VMEM((2, CH), jnp.int32),
            pltpu.VMEM((2, CH), jnp.int32),
            pltpu.VMEM((CH, D), jnp.float32),
            pltpu.VMEM((CH, D), jnp.float32),
            pltpu.VMEM_SHARED((NACC, D), jnp.float32),
            pltpu.SemaphoreType.DMA,
            pltpu.SemaphoreType.DMA,
            pltpu.SemaphoreType.DMA,
            pltpu.SemaphoreType.DMA,
        ],
    )
    return k(ts, echunks)


def _dinvpre_call(degpT, x, W1):
    def body(d_ref, x_ref, w_ref, dv_ref, ts_ref):
        deg = jnp.sum(d_ref[...], axis=1, keepdims=True) + 1.0
        dv = jnp.broadcast_to(lax.rsqrt(deg), (BR, D))
        dv_ref[...] = dv
        t = jnp.dot(x_ref[...], w_ref[...], preferred_element_type=jnp.float32)
        ts_ref[...] = t * dv
    return pl.pallas_call(
        body,
        out_shape=(jax.ShapeDtypeStruct((N, D), jnp.float32),
                   jax.ShapeDtypeStruct((N, D), jnp.float32)),
        grid=(N // BR,),
        in_specs=[pl.BlockSpec((BR, NW), lambda i: (i, 0)),
                  pl.BlockSpec((BR, D), lambda i: (i, 0)),
                  pl.BlockSpec((D, D), lambda i: (0, 0))],
        out_specs=(pl.BlockSpec((BR, D), lambda i: (i, 0)),
                   pl.BlockSpec((BR, D), lambda i: (i, 0))),
    )(degpT, x, W1)


def _postpre_call(p0, p1, ts, dinvb, b2d, res, Wn):
    spec = pl.BlockSpec((BR, D), lambda i: (i, 0))
    if res is None:
        def body(p0r, p1r, tsr, dvr, br, wr, hr, tr):
            dv = dvr[...]
            h = jnp.maximum((p0r[...] + p1r[...] + tsr[...]) * dv + br[...],
                            0.0)
            hr[...] = h
            tr[...] = jnp.dot(h, wr[...],
                              preferred_element_type=jnp.float32) * dv
        args = (p0, p1, ts, dinvb, b2d, Wn)
        extra = []
    else:
        def body(p0r, p1r, tsr, dvr, br, rr, wr, hr, tr):
            dv = dvr[...]
            h = jnp.maximum((p0r[...] + p1r[...] + tsr[...]) * dv + br[...],
                            0.0) + rr[...]
            hr[...] = h
            tr[...] = jnp.dot(h, wr[...],
                              preferred_element_type=jnp.float32) * dv
        args = (p0, p1, ts, dinvb, b2d, res, Wn)
        extra = [spec]
    return pl.pallas_call(
        body,
        out_shape=(jax.ShapeDtypeStruct((N, D), jnp.float32),
                   jax.ShapeDtypeStruct((N, D), jnp.float32)),
        grid=(N // BR,),
        in_specs=[spec, spec, spec, spec,
                  pl.BlockSpec((1, D), lambda i: (0, 0))] + extra
                 + [pl.BlockSpec((D, D), lambda i: (0, 0))],
        out_specs=(spec, spec),
    )(*args)


def _post_call(p0, p1, ts, dinvb, b2d, res):
    def body(p0r, p1r, tsr, dvr, br, rr, o_ref):
        s = (p0r[...] + p1r[...] + tsr[...]) * dvr[...] + br[...]
        o_ref[...] = jnp.maximum(s, 0.0) + rr[...]
    spec = pl.BlockSpec((BR, D), lambda i: (i, 0))
    return pl.pallas_call(
        body,
        out_shape=jax.ShapeDtypeStruct((N, D), jnp.float32),
        grid=(N // BR,),
        in_specs=[spec, spec, spec, spec,
                  pl.BlockSpec((1, D), lambda i: (0, 0)), spec],
        out_specs=spec,
    )(p0, p1, ts, dinvb, b2d, res)


RW = 312


def _segpool_body(h_hbm, b_hbm, sums, maxs, cnts,
                  bvec, rbuf, sacc, macc, cacc):
    w = lax.axis_index("c") * NS + lax.axis_index("s")
    rbase = w * RW
    pltpu.sync_copy(b_hbm.at[pl.ds(rbase, RW)], bvec.at[pl.ds(0, RW)])
    pltpu.sync_copy(h_hbm.at[pl.ds(rbase, RW)], rbuf.at[pl.ds(0, RW)])

    @pl.when(w == NW - 1)
    def _():
        pltpu.sync_copy(b_hbm.at[pl.ds(NW * RW, 16)],
                        bvec.at[pl.ds(RW, 16)])
        pltpu.sync_copy(h_hbm.at[pl.ds(NW * RW, 16)],
                        rbuf.at[pl.ds(RW, 16)])

    zero16 = jnp.zeros((16,), jnp.float32)
    ninf16 = jnp.full((16,), -jnp.inf, jnp.float32)

    def init(g, _):
        for j in range(8):
            sacc[g, pl.ds(j * 16, 16)] = zero16
            macc[g, pl.ds(j * 16, 16)] = ninf16
        cacc[g, pl.ds(0, 16)] = zero16
        return 0
    lax.fori_loop(0, G, init, 0)

    inv16 = jnp.full((16,), 1.0 / 16.0, jnp.float32)

    def row(r, _):
        b = bvec[pl.ds(r, 16)][0]
        for j in range(8):
            v = rbuf[r, pl.ds(j * 16, 16)]
            sacc[b, pl.ds(j * 16, 16)] = sacc[b, pl.ds(j * 16, 16)] + v
            macc[b, pl.ds(j * 16, 16)] = jnp.maximum(
                macc[b, pl.ds(j * 16, 16)], v)
        cacc[b, pl.ds(0, 16)] = cacc[b, pl.ds(0, 16)] + inv16
        return 0
    lax.fori_loop(0, RW, row, 0)

    @pl.when(w == NW - 1)
    def _():
        lax.fori_loop(RW, RW + 16, row, 0)

    pltpu.sync_copy(sacc, sums.at[w])
    pltpu.sync_copy(macc, maxs.at[w])
    pltpu.sync_copy(cacc, cnts.at[w])


def _segpool_call(h, batch):
    k = pl.kernel(
        _segpool_body,
        out_type=(jax.ShapeDtypeStruct((NW, G, D), jnp.float32),
                  jax.ShapeDtypeStruct((NW, G, D), jnp.float32),
                  jax.ShapeDtypeStruct((NW, G, 16), jnp.float32)),
        mesh=_mesh,
        scratch_types=[
            pltpu.VMEM((RW + 48,), jnp.int32),
            pltpu.VMEM((RW + 16, D), jnp.float32),
            pltpu.VMEM((G, D), jnp.float32),
            pltpu.VMEM((G, D), jnp.float32),
            pltpu.VMEM((G, 16), jnp.float32),
        ],
    )
    return k(h, batch)


def _mlp_call(sums, maxs, cnts, fW1, fb1, fW2, fb2, fW3, fb3):
    def body(s_ref, m_ref, c_ref, w1_ref, c1_ref, w2_ref, c2_ref,
             w3_ref, c3_ref, o_ref):
        def comb(i, carry):
            s, m, c = carry
            return (s + s_ref[i], jnp.maximum(m, m_ref[i]), c + c_ref[i])
        S, Mx, C = lax.fori_loop(
            0, NW, comb,
            (jnp.zeros((G, D), jnp.float32),
             jnp.full((G, D), -jnp.inf, jnp.float32),
             jnp.zeros((G, 16), jnp.float32)))
        cnt = jnp.sum(C, axis=1, keepdims=True)
        mean = S / jnp.maximum(cnt, 1.0)

        w1 = w1_ref[...]
        a = jnp.dot(mean, w1[:D, :], preferred_element_type=jnp.float32)
        a = a + jnp.dot(Mx, w1[D:, :], preferred_element_type=jnp.float32)
        a = jnp.maximum(a + c1_ref[...], 0.0)
        a = jnp.maximum(
            jnp.dot(a, w2_ref[...], preferred_element_type=jnp.float32)
            + c2_ref[...], 0.0)
        o_ref[...] = (jnp.dot(a, w3_ref[...], preferred_element_type=jnp.float32)
                      + c3_ref[...])

    return pl.pallas_call(
        body,
        out_shape=jax.ShapeDtypeStruct((G, 1), jnp.float32),
    )(sums, maxs, cnts, fW1, fb1, fW2, fb2, fW3, fb3)


def kernel(x, edge_index, batch, W1, b1, W2, b2, W3, b3,
           fW1, fb1, fW2, fb2, fW3, fb3):
    dst = edge_index[1]

    ppw = EPW - EPW_REAL
    pad_src = jnp.broadcast_to(jnp.arange(ppw, dtype=jnp.int32), (NW, ppw))
    pad_dst = jnp.broadcast_to(N + (jnp.arange(ppw, dtype=jnp.int32) % CH),
                               (NW, ppw))
    pad = jnp.stack([pad_src, pad_dst])
    echunks = (jnp.concatenate([edge_index.reshape(2, NW, EPW_REAL), pad],
                               axis=2)
               .reshape(2, NW, NCH, CH).transpose(1, 2, 0, 3)
               .reshape(NW * NCH, 2, CH))

    degp = _deg_call(dst)
    dinvb, ts1 = _dinvpre_call(degp.T, x, W1)
    p0, p1 = _msg_call(ts1, echunks)
    h1, ts2 = _postpre_call(p0, p1, ts1, dinvb, b1.reshape(1, D), None, W2)
    p0, p1 = _msg_call(ts2, echunks)
    h2, ts3 = _postpre_call(p0, p1, ts2, dinvb, b2.reshape(1, D), h1, W3)
    p0, p1 = _msg_call(ts3, echunks)
    h3 = _post_call(p0, p1, ts3, dinvb, b3.reshape(1, D), h2)

    sums, maxs, cnts = _segpool_call(h3, batch)
    return _mlp_call(sums, maxs, cnts, fW1, fb1.reshape(1, D),
                     fW2, fb2.reshape(1, D // 2), fW3, fb3.reshape(1, 1))
